# baseline (speedup 1.0000x reference)
import functools

import jax
import jax.numpy as jnp
from jax import lax
from jax.experimental import pallas as pl
from jax.experimental.pallas import tpu as pltpu
from jax.experimental.pallas import tpu_sc as plsc

B, S, D = 4, 8192, 768
NC, NS = 2, 16
NW = NC * NS
S_PER_W = S // NW
CHUNK = 32
N_CHUNK = S_PER_W // CHUNK
RING = 4


def _sc_body(x_hbm, t_hbm, o_hbm, x_v, t_v, in_sem, out_sem, t_sem):
    wid = lax.axis_index("s") * NC + lax.axis_index("c")
    row_base = wid * S_PER_W

    def rows(g):
        return pl.ds(row_base + g * CHUNK, CHUNK)

    def start_in(g, b, buf):
        pltpu.make_async_copy(
            x_hbm.at[b, rows(g), :], x_v.at[buf], in_sem
        ).start()

    def wait_in(buf):
        pltpu.make_async_copy(
            x_hbm.at[0, rows(0), :], x_v.at[buf], in_sem
        ).wait()

    def start_out(g, b, buf):
        pltpu.make_async_copy(
            x_v.at[buf], o_hbm.at[b, rows(g), :], out_sem
        ).start()

    def wait_out(buf):
        pltpu.make_async_copy(
            x_v.at[buf], o_hbm.at[0, rows(0), :], out_sem
        ).wait()

    def start_t(g):
        pltpu.make_async_copy(t_hbm.at[rows(g), :], t_v, t_sem).start()

    def wait_t():
        pltpu.make_async_copy(t_hbm.at[rows(0), :], t_v, t_sem).wait()

    start_t(0)
    start_in(0, 0, 0)
    start_in(0, 1, 1)

    def group_body(g, carry):
        wait_t()

        for b in range(B):
            buf = b
            wait_in(buf)

            if b < 2:
                @pl.when(g > 0)
                def _():
                    wait_out(b + 2)

                start_in(g, b + 2, b + 2)
            else:
                wait_out(b - 2)

                @pl.when(g < N_CHUNK - 1)
                def _():
                    start_in(g + 1, b - 2, b - 2)

            xk = x_v.at[buf]

            def row_body(r, c, xk=xk):
                r0 = r * 2
                for dr in range(2):
                    xr = xk.at[r0 + dr]
                    tr = t_v.at[r0 + dr]
                    for off in range(0, D, 16):
                        xr[pl.ds(off, 16)] = (
                            xr[pl.ds(off, 16)] + tr[pl.ds(off, 16)]
                        )
                return c

            lax.fori_loop(0, CHUNK // 2, row_body, 0)

            if b == B - 1:
                @pl.when(g < N_CHUNK - 1)
                def _():
                    start_t(g + 1)

            start_out(g, b, buf)
        return carry

    lax.fori_loop(0, N_CHUNK, group_body, 0)

    wait_out(2)
    wait_out(3)


@functools.partial(jax.jit)
def _sc_add(x, t):
    mesh = plsc.VectorSubcoreMesh(core_axis_name="c", subcore_axis_name="s")
    return pl.kernel(
        _sc_body,
        mesh=mesh,
        out_type=jax.ShapeDtypeStruct((B, S, D), jnp.float32),
        scratch_types=[
            pltpu.VMEM((RING, CHUNK, D), jnp.float32),
            pltpu.VMEM((CHUNK, D), jnp.float32),
            pltpu.SemaphoreType.DMA,
            pltpu.SemaphoreType.DMA,
            pltpu.SemaphoreType.DMA,
        ],
    )(x, t)


def kernel(inputs, table):
    return _sc_add(inputs, table)


# speedup vs baseline: 1.4371x; 1.4371x over previous
import functools

import jax
import jax.numpy as jnp
from jax import lax
from jax.experimental import pallas as pl
from jax.experimental.pallas import tpu as pltpu
from jax.experimental.pallas import tpu_sc as plsc

B, S, D = 4, 8192, 768
NC, NS = 2, 16
NW = NC * NS
S_PER_W = S // NW
CHUNK = 32
N_CHUNK = S_PER_W // CHUNK
RING = 4


def _sc_body(x_hbm, t_hbm, o_hbm, x_v, t_v, in_sem, out_sem, t_sem):
    wid = lax.axis_index("s") * NC + lax.axis_index("c")
    row_base = wid * S_PER_W

    def rows(g):
        return pl.ds(row_base + g * CHUNK, CHUNK)

    def start_in(g, b, buf):
        pltpu.make_async_copy(
            x_hbm.at[b, rows(g), :], x_v.at[buf], in_sem
        ).start()

    def wait_in(buf):
        pltpu.make_async_copy(
            x_hbm.at[0, rows(0), :], x_v.at[buf], in_sem
        ).wait()

    def start_out(g, b, buf):
        pltpu.make_async_copy(
            x_v.at[buf], o_hbm.at[b, rows(g), :], out_sem
        ).start()

    def wait_out(buf):
        pltpu.make_async_copy(
            x_v.at[buf], o_hbm.at[0, rows(0), :], out_sem
        ).wait()

    def start_t(g):
        pltpu.make_async_copy(t_hbm.at[rows(g), :], t_v, t_sem).start()

    def wait_t():
        pltpu.make_async_copy(t_hbm.at[rows(0), :], t_v, t_sem).wait()

    start_t(0)
    start_in(0, 0, 0)
    start_in(0, 1, 1)

    def group_body(g, carry):
        wait_t()

        for b in range(B):
            buf = b
            wait_in(buf)

            if b < 2:
                @pl.when(g > 0)
                def _():
                    wait_out(b + 2)

                start_in(g, b + 2, b + 2)
            else:
                wait_out(b - 2)

                @pl.when(g < N_CHUNK - 1)
                def _():
                    start_in(g + 1, b - 2, b - 2)

            xk = x_v.at[buf]

            def row_body(r, c, xk=xk):
                xr = xk.at[r]
                tr = t_v.at[r]
                for off in range(0, D, 16):
                    xr[pl.ds(off, 16)] = xr[pl.ds(off, 16)] + tr[pl.ds(off, 16)]
                return c

            lax.fori_loop(0, CHUNK, row_body, 0)

            if b == B - 1:
                @pl.when(g < N_CHUNK - 1)
                def _():
                    start_t(g + 1)

            start_out(g, b, buf)
        return carry

    lax.fori_loop(0, N_CHUNK, group_body, 0)

    wait_out(2)
    wait_out(3)


@functools.partial(jax.jit)
def _sc_add(x, t):
    mesh = plsc.VectorSubcoreMesh(core_axis_name="c", subcore_axis_name="s")
    return pl.kernel(
        _sc_body,
        mesh=mesh,
        out_type=jax.ShapeDtypeStruct((B, S, D), jnp.float32),
        scratch_types=[
            pltpu.VMEM((RING, CHUNK, D), jnp.float32),
            pltpu.VMEM((CHUNK, D), jnp.float32),
            pltpu.SemaphoreType.DMA,
            pltpu.SemaphoreType.DMA,
            pltpu.SemaphoreType.DMA,
        ],
    )(x, t)


def kernel(inputs, table):
    return _sc_add(inputs, table)
